# Initial kernel scaffold; baseline (speedup 1.0000x reference)
import jax
import jax.numpy as jnp
from jax.experimental import pallas as pl


def kernel(positions, tables):
    raise NotImplementedError("write your pallas kernel here")


# speedup vs baseline: 26.0088x; 26.0088x over previous
import functools

import jax
import jax.numpy as jnp
import numpy as np
from jax import lax
from jax.experimental import pallas as pl
from jax.experimental.pallas import tpu as pltpu
from jax.experimental.pallas import tpu_sc as plsc

N_POINTS = 524288
NUM_LEVELS = 16
TBL = 4096
LANES = 16
NW = 32
LV_GROUPS = 2
LV_PER_G = NUM_LEVELS // LV_GROUPS
CHUNK = N_POINTS // (NW // LV_GROUPS)
BLK = 2048
P1 = np.int32(np.uint32(2654435761))
P2 = np.int32(np.uint32(805459861))

_mesh = plsc.VectorSubcoreMesh(core_axis_name="c", subcore_axis_name="s")


@functools.partial(
    pl.kernel,
    mesh=_mesh,
    compiler_params=pltpu.CompilerParams(use_tc_tiling_on_sc=False,
                                         needs_layout_passes=False),
    out_type=jax.ShapeDtypeStruct((N_POINTS, NUM_LEVELS * 2), jnp.float32),
    scratch_types=[
        pltpu.VMEM((LV_PER_G * TBL * 2,), jnp.float32),
        pltpu.VMEM((BLK, 3), jnp.float32),
        pltpu.VMEM((BLK, LV_PER_G * 2), jnp.float32),
    ],
)
def _encode_sc(tab_hbm, pos_hbm, out_hbm, tab_v, pos_v, out_v):
    wid = lax.axis_index("s") * 2 + lax.axis_index("c")
    grp = wid & 1
    chunk = wid >> 1

    half = LV_PER_G * TBL
    i32 = jnp.int32
    pltpu.sync_copy(tab_hbm.at[i32(0), pl.ds(grp * i32(half), half)],
                    tab_v.at[pl.ds(i32(0), half)])
    pltpu.sync_copy(tab_hbm.at[i32(1), pl.ds(grp * i32(half), half)],
                    tab_v.at[pl.ds(i32(half), half)])

    iota = lax.iota(jnp.int32, LANES)
    one = jnp.float32(1.0)

    grp0 = grp == 0
    a2s, rm1s = [], []
    for l in range(LV_PER_G):
        a2s.append(jnp.where(grp0, jnp.float32(2.0 ** (l + 3)),
                             jnp.float32(2.0 ** (l + LV_PER_G + 3))))
        rm1s.append(jnp.where(grp0, jnp.int32(16 * 2 ** l - 1),
                              jnp.int32(16 * 2 ** (l + LV_PER_G) - 1)))

    def block_body(t, _):
        base = chunk * i32(CHUNK) + t * i32(BLK)
        pltpu.sync_copy(pos_hbm.at[pl.ds(base, BLK), :], pos_v)

        def group_body(j, _):
            rows = j * i32(LANES) + iota
            x = plsc.load_gather(pos_v, [rows, jnp.full((LANES,), 0, jnp.int32)])
            y = plsc.load_gather(pos_v, [rows, jnp.full((LANES,), 1, jnp.int32)])
            z = plsc.load_gather(pos_v, [rows, jnp.full((LANES,), 2, jnp.int32)])
            ppx, ppy, ppz = x + one, y + one, z + one
            phx, phy, phz = ppx * 0.5, ppy * 0.5, ppz * 0.5

            for l in range(LV_PER_G):
                a2 = a2s[l]
                rm1 = rm1s[l]
                lb = jnp.int32(l * TBL)

                def axis(pp, ph):
                    A = pp * a2
                    scaled = A - ph
                    ti = scaled.astype(jnp.int32)
                    tf = ti.astype(jnp.float32)
                    w = scaled - tf
                    c1 = jnp.minimum(ti + 1, rm1)
                    return ti, c1, w

                cx0, cx1, wx = axis(ppx, phx)
                cy0, cy1, wy = axis(ppy, phy)
                cz0, cz1, wz = axis(ppz, phz)

                mx0 = (cx0 & 4095) | lb
                mx1 = (cx1 & 4095) | lb
                my0 = (cy0 * P1) & 4095
                my1 = (cy1 * P1) & 4095
                mz0 = (cz0 * P2) & 4095
                mz1 = (cz1 * P2) & 4095

                f = []
                for mx in (mx0, mx1):
                    for my in (my0, my1):
                        for mz in (mz0, mz1):
                            i0 = mx ^ my ^ mz
                            f.append((plsc.load_gather(tab_v, [i0]),
                                      plsc.load_gather(tab_v, [i0 + half])))

                omx, omy, omz = one - wx, one - wy, one - wz
                res = []
                for k in range(2):
                    c00 = f[0][k] * omx + f[1][k] * wx
                    c01 = f[2][k] * omx + f[3][k] * wx
                    c10 = f[4][k] * omx + f[5][k] * wx
                    c11 = f[6][k] * omx + f[7][k] * wx
                    d0 = c00 * omy + c01 * wy
                    d1 = c10 * omy + c11 * wy
                    res.append(d0 * omz + d1 * wz)

                plsc.store_scatter(out_v, [rows, jnp.full((LANES,), 2 * l, jnp.int32)], res[0])
                plsc.store_scatter(out_v, [rows, jnp.full((LANES,), 2 * l + 1, jnp.int32)], res[1])
            return i32(0)

        lax.fori_loop(i32(0), i32(BLK // LANES), group_body, i32(0))
        pltpu.sync_copy(out_v,
                        out_hbm.at[pl.ds(base, BLK),
                                   pl.ds(grp * i32(LV_PER_G * 2), LV_PER_G * 2)])
        return i32(0)

    lax.fori_loop(i32(0), i32(CHUNK // BLK), block_body, i32(0))


def kernel(positions, tables):
    tabp = jnp.transpose(tables[:, :TBL, :], (2, 0, 1)).reshape(2, NUM_LEVELS * TBL)
    tabp = tabp.astype(jnp.float32)
    return _encode_sc(tabp, positions.astype(jnp.float32))


# speedup vs baseline: 26.3015x; 1.0113x over previous
import functools

import jax
import jax.numpy as jnp
import numpy as np
from jax import lax
from jax.experimental import pallas as pl
from jax.experimental.pallas import tpu as pltpu
from jax.experimental.pallas import tpu_sc as plsc

N_POINTS = 524288
NUM_LEVELS = 16
TBL = 4096
LANES = 16
NW = 32
LV_GROUPS = 2
LV_PER_G = NUM_LEVELS // LV_GROUPS
HALF = LV_PER_G * TBL
CHUNK = N_POINTS // (NW // LV_GROUPS)
BLK = 2048
GW = LV_PER_G * 2
P1 = np.int32(np.uint32(2654435761))
P2 = np.int32(np.uint32(805459861))

_mesh = plsc.VectorSubcoreMesh(core_axis_name="c", subcore_axis_name="s")


@functools.partial(
    pl.kernel,
    mesh=_mesh,
    compiler_params=pltpu.CompilerParams(needs_layout_passes=False),
    out_type=jax.ShapeDtypeStruct((LV_GROUPS * N_POINTS * GW,), jnp.float32),
    scratch_types=[
        pltpu.VMEM((LV_GROUPS * HALF,), jnp.float32),
        pltpu.VMEM((BLK * 3,), jnp.float32),
        pltpu.VMEM((BLK * GW,), jnp.float32),
    ],
)
def _encode_sc(tab_hbm, pos_hbm, out_hbm, tab_v, pos_v, out_v):
    i32 = jnp.int32
    wid = lax.axis_index("s") * 2 + lax.axis_index("c")
    grp = wid & 1
    chunk = wid >> 1

    pltpu.sync_copy(tab_hbm.at[pl.ds(grp * i32(HALF), HALF)],
                    tab_v.at[pl.ds(i32(0), HALF)])
    pltpu.sync_copy(tab_hbm.at[pl.ds(grp * i32(HALF) + i32(2 * HALF), HALF)],
                    tab_v.at[pl.ds(i32(HALF), HALF)])

    iota = lax.iota(jnp.int32, LANES)
    one = jnp.float32(1.0)

    grp0 = grp == 0
    a2s, rm1s = [], []
    for l in range(LV_PER_G):
        a2s.append(jnp.where(grp0, jnp.float32(2.0 ** (l + 3)),
                             jnp.float32(2.0 ** (l + LV_PER_G + 3))))
        rm1s.append(jnp.where(grp0, jnp.int32(16 * 2 ** l - 1),
                              jnp.int32(16 * 2 ** (l + LV_PER_G) - 1)))

    def block_body(t, _):
        base = chunk * i32(CHUNK) + t * i32(BLK)
        pltpu.sync_copy(pos_hbm.at[pl.ds(base * 3, BLK * 3)], pos_v)

        def group_body(j, _):
            rows = j * i32(LANES) + iota
            rows3 = rows * 3
            x = plsc.load_gather(pos_v, [rows3])
            y = plsc.load_gather(pos_v, [rows3 + 1])
            z = plsc.load_gather(pos_v, [rows3 + 2])
            ppx, ppy, ppz = x + one, y + one, z + one
            phx, phy, phz = ppx * 0.5, ppy * 0.5, ppz * 0.5
            rowcol = rows * i32(GW)

            for l in range(LV_PER_G):
                a2 = a2s[l]
                rm1 = rm1s[l]
                lb = i32(l * TBL)

                def axis(pp, ph):
                    A = pp * a2
                    scaled = A - ph
                    ti = scaled.astype(jnp.int32)
                    tf = ti.astype(jnp.float32)
                    w = scaled - tf
                    c1 = jnp.minimum(ti + 1, rm1)
                    return ti, c1, w

                cx0, cx1, wx = axis(ppx, phx)
                cy0, cy1, wy = axis(ppy, phy)
                cz0, cz1, wz = axis(ppz, phz)

                mx0 = (cx0 & 4095) | lb
                mx1 = (cx1 & 4095) | lb
                my0 = (cy0 * P1) & 4095
                my1 = (cy1 * P1) & 4095
                mz0 = (cz0 * P2) & 4095
                mz1 = (cz1 * P2) & 4095

                f = []
                for mx in (mx0, mx1):
                    for my in (my0, my1):
                        for mz in (mz0, mz1):
                            i0 = mx ^ my ^ mz
                            f.append((plsc.load_gather(tab_v, [i0]),
                                      plsc.load_gather(tab_v, [i0 + i32(HALF)])))

                omx, omy, omz = one - wx, one - wy, one - wz
                res = []
                for k in range(2):
                    c00 = f[0][k] * omx + f[1][k] * wx
                    c01 = f[2][k] * omx + f[3][k] * wx
                    c10 = f[4][k] * omx + f[5][k] * wx
                    c11 = f[6][k] * omx + f[7][k] * wx
                    d0 = c00 * omy + c01 * wy
                    d1 = c10 * omy + c11 * wy
                    res.append(d0 * omz + d1 * wz)

                plsc.store_scatter(out_v, [rowcol + i32(2 * l)], res[0])
                plsc.store_scatter(out_v, [rowcol + i32(2 * l + 1)], res[1])
            return i32(0)

        lax.fori_loop(i32(0), i32(BLK // LANES), group_body, i32(0))
        pltpu.sync_copy(out_v,
                        out_hbm.at[pl.ds(grp * i32(N_POINTS * GW) + base * i32(GW),
                                         BLK * GW)])
        return i32(0)

    lax.fori_loop(i32(0), i32(CHUNK // BLK), block_body, i32(0))


def kernel(positions, tables):
    tabp = jnp.transpose(tables[:, :TBL, :], (2, 0, 1)).reshape(-1)
    tabp = tabp.astype(jnp.float32)
    flat = _encode_sc(tabp, positions.astype(jnp.float32).reshape(-1))
    return (flat.reshape(LV_GROUPS, N_POINTS, GW)
                .transpose(1, 0, 2)
                .reshape(N_POINTS, NUM_LEVELS * 2))
